# baseline (speedup 1.0000x reference)
import functools

import jax
import jax.numpy as jnp
import numpy as np
from jax import lax
from jax.experimental import pallas as pl
from jax.experimental.pallas import tpu as pltpu
from jax.experimental.pallas import tpu_sc as plsc

_LIST_OF_N = [100003, 100019, 100043, 100049, 100057, 100069, 100103, 100109,
              100129, 100151, 100153, 100169]
_D = 64
_H = len(_LIST_OF_N)

_NC, _NS, _L = 2, 16, 16
_NW = _NC * _NS

_CHUNK = 768
_SUB = 128
_NSUB = _CHUNK // _SUB


def _build_sc_gather(total, n_chunks, per_w):
    mesh = plsc.VectorSubcoreMesh(core_axis_name="c", subcore_axis_name="s")

    @functools.partial(
        pl.kernel,
        mesh=mesh,
        out_type=jax.ShapeDtypeStruct((total, _D), jnp.float32),
        scratch_types=[
            pltpu.VMEM((_CHUNK,), jnp.int32),
            pltpu.VMEM((_CHUNK,), jnp.int32),
            pltpu.VMEM((_CHUNK, _D), jnp.float32),
            pltpu.SemaphoreType.DMA,
        ],
        compiler_params=pltpu.CompilerParams(use_tc_tiling_on_sc=False),
    )
    def sc_gather(ids_hbm, offs_hbm, table_hbm, out_hbm,
                  idx_v, offs_v, rows_v, sem):
        wid = lax.axis_index("s") * _NC + lax.axis_index("c")
        pltpu.sync_copy(offs_hbm, offs_v)
        base0 = wid * per_w
        for ci in range(n_chunks):
            base = base0 + ci * _CHUNK
            pltpu.sync_copy(ids_hbm.at[pl.ds(base, _CHUNK)], idx_v)
            for l in range(_CHUNK // _L):
                sl = pl.ds(l * _L, _L)
                idx_v[sl] = idx_v[sl] + offs_v[sl]
            copies = []
            for j in range(_NSUB):
                copies.append(pltpu.async_copy(
                    table_hbm.at[idx_v.at[pl.ds(j * _SUB, _SUB)]],
                    rows_v.at[pl.ds(j * _SUB, _SUB)],
                    sem))
            for c in copies:
                c.wait()
            pltpu.sync_copy(rows_v, out_hbm.at[pl.ds(base, _CHUNK)])

    return sc_gather


def kernel(input_ids, table):
    B, T, H = input_ids.shape
    total = B * T * H
    per_w = total // _NW
    n_chunks = per_w // _CHUNK

    offsets = np.concatenate([[0], np.cumsum(_LIST_OF_N[:-1])]).astype(np.int32)
    flat = np.arange(_CHUNK) % _H
    offs_pat = jnp.asarray(offsets[flat])

    ids_flat = input_ids.reshape(total)
    sc_gather = _build_sc_gather(total, n_chunks, per_w)
    out = sc_gather(ids_flat, offs_pat, table)
    return out.reshape(B, T, H, _D)


# speedup vs baseline: 1.0556x; 1.0556x over previous
import functools

import jax
import jax.numpy as jnp
import numpy as np
from jax import lax
from jax.experimental import pallas as pl
from jax.experimental.pallas import tpu as pltpu
from jax.experimental.pallas import tpu_sc as plsc

_LIST_OF_N = [100003, 100019, 100043, 100049, 100057, 100069, 100103, 100109,
              100129, 100151, 100153, 100169]
_D = 64
_DP = 128
_H = len(_LIST_OF_N)

_NC, _NS, _L = 2, 16, 16
_NW = _NC * _NS

_CHUNK = 768
_SUB = 128
_NSUB = _CHUNK // _SUB


def _build_sc_gather(total, n_chunks, per_w):
    mesh = plsc.VectorSubcoreMesh(core_axis_name="c", subcore_axis_name="s")

    @functools.partial(
        pl.kernel,
        mesh=mesh,
        out_type=jax.ShapeDtypeStruct((total, _D), jnp.float32),
        scratch_types=[
            pltpu.VMEM((_CHUNK,), jnp.int32),
            pltpu.VMEM((_CHUNK,), jnp.int32),
            pltpu.VMEM((_CHUNK, _DP), jnp.float32),
            pltpu.SemaphoreType.DMA,
        ],
        compiler_params=pltpu.CompilerParams(use_tc_tiling_on_sc=False),
    )
    def sc_gather(ids_hbm, offs_hbm, table_hbm, out_hbm,
                  idx_v, offs_v, rows_v, sem):
        wid = lax.axis_index("s") * _NC + lax.axis_index("c")
        pltpu.sync_copy(offs_hbm, offs_v)
        base0 = wid * per_w
        for ci in range(n_chunks):
            base = base0 + ci * _CHUNK
            pltpu.sync_copy(ids_hbm.at[pl.ds(base, _CHUNK)], idx_v)
            for l in range(_CHUNK // _L):
                sl = pl.ds(l * _L, _L)
                idx_v[sl] = idx_v[sl] + offs_v[sl]
            copies = []
            for j in range(_NSUB):
                copies.append(pltpu.async_copy(
                    table_hbm.at[idx_v.at[pl.ds(j * _SUB, _SUB)]],
                    rows_v.at[pl.ds(j * _SUB, _SUB)],
                    sem))
            for c in copies:
                c.wait()
            pltpu.sync_copy(rows_v.at[:, pl.ds(0, _D)],
                            out_hbm.at[pl.ds(base, _CHUNK)])

    return sc_gather


def kernel(input_ids, table):
    B, T, H = input_ids.shape
    total = B * T * H
    per_w = total // _NW
    n_chunks = per_w // _CHUNK

    offsets = np.concatenate([[0], np.cumsum(_LIST_OF_N[:-1])]).astype(np.int32)
    flat = np.arange(_CHUNK) % _H
    offs_pat = jnp.asarray(offsets[flat])

    table_p = jnp.pad(table, ((0, 0), (0, _DP - _D)))

    ids_flat = input_ids.reshape(total)
    sc_gather = _build_sc_gather(total, n_chunks, per_w)
    out = sc_gather(ids_flat, offs_pat, table_p)
    return out.reshape(B, T, H, _D)


# speedup vs baseline: 1.0988x; 1.0409x over previous
import functools

import jax
import jax.numpy as jnp
import numpy as np
from jax import lax
from jax.experimental import pallas as pl
from jax.experimental.pallas import tpu as pltpu
from jax.experimental.pallas import tpu_sc as plsc

_LIST_OF_N = [100003, 100019, 100043, 100049, 100057, 100069, 100103, 100109,
              100129, 100151, 100153, 100169]
_D = 64
_DP = 128
_H = len(_LIST_OF_N)

_NC, _NS, _L = 2, 16, 16
_NW = _NC * _NS

_CHUNK = 768
_SUB = 128
_NSUB = _CHUNK // _SUB


def _build_sc_gather(total, n_chunks, per_w):
    mesh = plsc.VectorSubcoreMesh(core_axis_name="c", subcore_axis_name="s")

    @functools.partial(
        pl.kernel,
        mesh=mesh,
        out_type=jax.ShapeDtypeStruct((total, _D), jnp.float32),
        scratch_types=[
            pltpu.VMEM((_CHUNK,), jnp.int32),
            pltpu.VMEM((_CHUNK,), jnp.int32),
            pltpu.VMEM((_CHUNK, _D), jnp.float32),
            pltpu.SemaphoreType.DMA,
        ],
        compiler_params=pltpu.CompilerParams(use_tc_tiling_on_sc=False),
    )
    def sc_gather(ids_hbm, offs_hbm, table_hbm, out_hbm,
                  idx_v, offs_v, rows_v, sem):
        wid = lax.axis_index("s") * _NC + lax.axis_index("c")
        pltpu.sync_copy(offs_hbm, offs_v)
        base0 = wid * per_w
        for ci in range(n_chunks):
            base = base0 + ci * _CHUNK
            pltpu.sync_copy(ids_hbm.at[pl.ds(base, _CHUNK)], idx_v)
            for l in range(_CHUNK // _L):
                sl = pl.ds(l * _L, _L)
                v = idx_v[sl] + offs_v[sl]
                idx_v[sl] = v + v
            copies = []
            for j in range(_NSUB):
                copies.append(pltpu.async_copy(
                    table_hbm.at[idx_v.at[pl.ds(j * _SUB, _SUB)]],
                    rows_v.at[pl.ds(j * _SUB, _SUB)],
                    sem))
            for c in copies:
                c.wait()
            pltpu.sync_copy(rows_v, out_hbm.at[pl.ds(base, _CHUNK)])

    return sc_gather


def kernel(input_ids, table):
    B, T, H = input_ids.shape
    total = B * T * H
    per_w = total // _NW
    n_chunks = per_w // _CHUNK

    offsets = np.concatenate([[0], np.cumsum(_LIST_OF_N[:-1])]).astype(np.int32)
    flat = np.arange(_CHUNK) % _H
    offs_pat = jnp.asarray(offsets[flat])

    table_p = jnp.pad(table, ((0, 0), (0, _DP - _D)))
    table_v = table_p.reshape(table.shape[0] * 2, _D)

    ids_flat = input_ids.reshape(total)
    sc_gather = _build_sc_gather(total, n_chunks, per_w)
    out = sc_gather(ids_flat, offs_pat, table_v)
    return out.reshape(B, T, H, _D)
